# baseline (speedup 1.0000x reference)
import functools

import jax
import jax.numpy as jnp
from jax import lax
from jax.experimental import pallas as pl
from jax.experimental.pallas import tpu as pltpu
from jax.experimental.pallas import tpu_sc as plsc

B = 4096
L = 200
N = B * L
WORD_DIM = 64
WORD_PAD = 128
FEAT_LEN = 512
FEAT_DIM = 16
OUT_DIM = 96

NC, NS = 2, 16
NW = NC * NS
SUB = 128
JT = OUT_DIM // 8


def _sc_embed(xT, lT, rT, wv, p1, p2):
    mesh = plsc.VectorSubcoreMesh(core_axis_name="c", subcore_axis_name="s")

    @functools.partial(
        pl.kernel,
        mesh=mesh,
        out_type=jax.ShapeDtypeStruct((L, JT, NW, 8, SUB), jnp.float32),
        compiler_params=pltpu.CompilerParams(needs_layout_passes=False),
        scratch_types=[
            pltpu.VMEM((SUB,), jnp.int32),
            pltpu.VMEM((SUB,), jnp.int32),
            pltpu.VMEM((SUB,), jnp.int32),
            pltpu.VMEM((SUB,), jnp.int32),
            pltpu.VMEM((SUB,), jnp.int32),
            pltpu.VMEM((SUB,), jnp.int32),
            pltpu.VMEM((SUB, WORD_PAD), jnp.float32),
            pltpu.VMEM((SUB, WORD_PAD), jnp.float32),
            pltpu.VMEM((SUB, WORD_PAD), jnp.float32),
            pltpu.VMEM((SUB, WORD_PAD), jnp.float32),
            pltpu.VMEM((JT, 8, SUB), jnp.float32),
            pltpu.VMEM((JT, 8, SUB), jnp.float32),
            pltpu.VMEM_SHARED((FEAT_LEN, WORD_PAD), jnp.float32),
            pltpu.VMEM_SHARED((FEAT_LEN, WORD_PAD), jnp.float32),
            pltpu.SemaphoreType.DMA,
            pltpu.SemaphoreType.DMA,
            pltpu.SemaphoreType.DMA,
            pltpu.SemaphoreType.DMA,
            pltpu.SemaphoreType.DMA,
            pltpu.SemaphoreType.DMA,
            pltpu.SemaphoreType.DMA,
        ],
    )
    def k(xh, lh, rh, wvh, p1h, p2h, outh,
          xi0, xi1, li0, li1, ri0, ri1,
          wvb0, wvb1, p1b, p2b, tb0, tb1, p1s, p2s,
          si0, si1, sg0, sg1, sp, so0, so1):
        cid = lax.axis_index("c")
        sid = lax.axis_index("s")
        bt = sid * NC + cid
        b0 = bt * SUB

        xi = (xi0, xi1)
        li = (li0, li1)
        ri = (ri0, ri1)
        wvb = (wvb0, wvb1)
        tb = (tb0, tb1)
        s_idx = (si0, si1)
        s_gat = (sg0, sg1)
        s_out = (so0, so1)

        pltpu.sync_copy(p1h, p1s)
        pltpu.sync_copy(p2h, p2s)

        def idx_cp(c, slot):
            s = s_idx[slot]
            return (pltpu.make_async_copy(xh.at[c, pl.ds(b0, SUB)], xi[slot], s),
                    pltpu.make_async_copy(lh.at[c, pl.ds(b0, SUB)], li[slot], s),
                    pltpu.make_async_copy(rh.at[c, pl.ds(b0, SUB)], ri[slot], s))

        def wv_cp(c, slot):
            return (pltpu.make_async_copy(wvh.at[xi[slot]], wvb[slot],
                                          s_gat[slot]),)

        def pf_cp(c, slot):
            return (pltpu.make_async_copy(p1s.at[li[slot]], p1b, sp),
                    pltpu.make_async_copy(p2s.at[ri[slot]], p2b, sp))

        def out_cp(c, slot):
            return (pltpu.make_async_copy(
                tb[slot], outh.at[c, :, bt, :, :], s_out[slot]),)

        def start(cs):
            for cp in cs:
                cp.start()

        def wait(cs):
            for cp in cs:
                cp.wait()

        lanes = lax.iota(jnp.int32, 16)
        rows_g = [lanes + (g * 16) for g in range(8)]

        def assemble(slot):
            dst, wb = tb[slot], wvb[slot]

            def wv_body(jt, carry):
                for ji in range(8):
                    col = jt * 8 + ji
                    for g in range(8):
                        v = plsc.load_gather(
                            wb, [rows_g[g], jnp.full((16,), col, jnp.int32)])
                        dst[jt, ji, pl.ds(g * 16, 16)] = v
                return carry

            lax.fori_loop(0, 8, wv_body, 0)

            for jt, src, base in ((8, p1b, 0), (9, p1b, 8),
                                  (10, p2b, 0), (11, p2b, 8)):
                for ji in range(8):
                    col = base + ji
                    for g in range(8):
                        v = plsc.load_gather(
                            src, [rows_g[g], jnp.full((16,), col, jnp.int32)])
                        dst[jt, ji, pl.ds(g * 16, 16)] = v

        start(idx_cp(0, 0))
        start(idx_cp(1, 1))
        wait(idx_cp(0, 0))
        start(wv_cp(0, 0))
        start(pf_cp(0, 0))

        def do_chunk(c, slot):
            @pl.when(c >= 2)
            def _():
                wait(out_cp(c - 2, slot))

            wait(wv_cp(c, slot))
            wait(pf_cp(c, slot))

            @pl.when(c + 1 < L)
            def _():
                wait(idx_cp(c + 1, slot ^ 1))
                start(wv_cp(c + 1, slot ^ 1))

            @pl.when(c + 2 < L)
            def _():
                start(idx_cp(c + 2, slot))

            assemble(slot)
            start(out_cp(c, slot))

            @pl.when(c + 1 < L)
            def _():
                start(pf_cp(c + 1, slot ^ 1))

        def body(i, carry):
            do_chunk(2 * i, 0)
            do_chunk(2 * i + 1, 1)
            return carry

        lax.fori_loop(0, L // 2, body, 0)
        wait(out_cp(L - 2, 0))
        wait(out_cp(L - 1, 1))

    return k(xT, lT, rT, wv, p1, p2)


def _pad128(t, block_rows):
    v, d = t.shape

    def body(in_ref, out_ref):
        out_ref[...] = jnp.concatenate(
            [in_ref[...], jnp.zeros((block_rows, WORD_PAD - d), jnp.float32)],
            axis=1)

    return pl.pallas_call(
        body,
        grid=(v // block_rows,),
        in_specs=[pl.BlockSpec((block_rows, d), lambda i: (i, 0))],
        out_specs=pl.BlockSpec((block_rows, WORD_PAD), lambda i: (i, 0)),
        out_shape=jax.ShapeDtypeStruct((v, WORD_PAD), jnp.float32),
    )(t)


def kernel(x, ldist, rdist, Wv, pf1, pf2):
    xT = jnp.swapaxes(x, 0, 1).astype(jnp.int32)
    lT = jnp.swapaxes(ldist, 0, 1).astype(jnp.int32)
    rT = jnp.swapaxes(rdist, 0, 1).astype(jnp.int32)
    wv128 = _pad128(Wv, 2000)
    p1128 = _pad128(pf1, 512)
    p2128 = _pad128(pf2, 512)
    out5 = _sc_embed(xT, lT, rT, wv128, p1128, p2128)
    out = out5.transpose(2, 4, 0, 1, 3).reshape(B, 1, L, OUT_DIM)
    return out
